# baseline (speedup 1.0000x reference)
import jax
import jax.numpy as jnp
from jax.experimental import pallas as pl

N_ATOMS = 100000
EMB = 128
N_MOL = 256
WIN = 128
BLOCK = 5000
N_STEPS = 10


def _accumulate_stream(e_ref, ids_ref, w_bf, b_val, out_ref):
    v = jnp.dot(e_ref[:].astype(jnp.bfloat16), w_bf,
                preferred_element_type=jnp.float32)
    v = (v + b_val).reshape(1, BLOCK)
    ids = ids_ref[0, 0, :]
    first = ids_ref[0, 0, 0]
    span = ids_ref[0, 0, BLOCK - 1] - first

    @pl.when(span < WIN)
    def _():
        idsw = ids - first
        colw = jax.lax.broadcasted_iota(jnp.int32, (BLOCK, WIN), 1)
        ohw = (idsw[:, None] == colw).astype(jnp.float32)
        cw = jax.lax.dot_general(
            v, ohw, (((1,), (0,)), ((), ())),
            preferred_element_type=jnp.float32)
        rows = jax.lax.broadcasted_iota(jnp.int32, (WIN, N_MOL), 0) + first
        cols = jax.lax.broadcasted_iota(jnp.int32, (WIN, N_MOL), 1)
        place = (rows == cols).astype(jnp.float32)
        out_ref[:] += jax.lax.dot_general(
            cw, place, (((1,), (0,)), ((), ())),
            preferred_element_type=jnp.float32)

    @pl.when(span >= WIN)
    def _():
        col = jax.lax.broadcasted_iota(jnp.int32, (BLOCK, N_MOL), 1)
        oh = (ids[:, None] == col).astype(jnp.float32)
        out_ref[:] += jax.lax.dot_general(
            v, oh, (((1,), (0,)), ((), ())),
            preferred_element_type=jnp.float32)


def _head_kernel(ea_ref, eb_ref, ia_ref, ib_ref, w_ref, b_ref, out_ref):
    i = pl.program_id(0)
    w_bf = w_ref[:].astype(jnp.bfloat16)
    b_val = b_ref[0, 0]

    @pl.when(i == 0)
    def _():
        out_ref[:] = jnp.zeros_like(out_ref)

    _accumulate_stream(ea_ref, ia_ref, w_bf, b_val, out_ref)
    _accumulate_stream(eb_ref, ib_ref, w_bf, b_val, out_ref)


def kernel(energy, batch, W, b):
    ids3d = batch.astype(jnp.int32).reshape(2 * N_STEPS, 1, BLOCK)
    b2d = b.reshape(1, 1)
    out = pl.pallas_call(
        _head_kernel,
        grid=(N_STEPS,),
        in_specs=[
            pl.BlockSpec((BLOCK, EMB), lambda i: (i, 0)),
            pl.BlockSpec((BLOCK, EMB), lambda i: (i + N_STEPS, 0)),
            pl.BlockSpec((1, 1, BLOCK), lambda i: (i, 0, 0)),
            pl.BlockSpec((1, 1, BLOCK), lambda i: (i + N_STEPS, 0, 0)),
            pl.BlockSpec((EMB, 1), lambda i: (0, 0)),
            pl.BlockSpec((1, 1), lambda i: (0, 0)),
        ],
        out_specs=pl.BlockSpec((1, N_MOL), lambda i: (0, 0)),
        out_shape=jax.ShapeDtypeStruct((1, N_MOL), jnp.float32),
    )(energy, energy, ids3d, ids3d, W, b2d)
    return out[0]
